# baseline (speedup 1.0000x reference)
import jax
import jax.numpy as jnp
from jax.experimental import pallas as pl
from jax.experimental.pallas import tpu as pltpu

N = 10000
D_IN = 128
D_OUT = 128
BR = 400
NR = N // BR


def _gcn_kernel(
    weight_ref,
    bias_ref,
    feat_hbm,
    adj_hbm,
    out_ref,
    buf0,
    buf1,
    feat_buf,
    support_ref,
    sem0,
    sem1,
    fsem,
):
    r = pl.program_id(0)

    @pl.when(r == 0)
    def _():
        fcp = pltpu.make_async_copy(feat_hbm, feat_buf, fsem)
        fcp.start()
        pltpu.make_async_copy(adj_hbm.at[pl.ds(0, BR), :], buf0, sem0).start()
        pltpu.make_async_copy(adj_hbm.at[pl.ds(BR, BR), :], buf1, sem1).start()
        fcp.wait()
        support_ref[...] = jnp.dot(
            feat_buf[...], weight_ref[...], preferred_element_type=jnp.float32
        ).astype(jnp.bfloat16)

    nxt = r + 1

    @pl.when((r >= 1) & (nxt < NR) & (nxt % 2 == 0))
    def _():
        pltpu.make_async_copy(adj_hbm.at[pl.ds(nxt * BR, BR), :], buf0, sem0).start()

    @pl.when((r >= 1) & (nxt < NR) & (nxt % 2 == 1))
    def _():
        pltpu.make_async_copy(adj_hbm.at[pl.ds(nxt * BR, BR), :], buf1, sem1).start()

    @pl.when(r % 2 == 0)
    def _():
        pltpu.make_async_copy(adj_hbm.at[pl.ds(r * BR, BR), :], buf0, sem0).wait()
        out_ref[...] = (
            jnp.dot(
                buf0[...].astype(jnp.bfloat16),
                support_ref[...],
                preferred_element_type=jnp.float32,
            )
            + bias_ref[...]
        )

    @pl.when(r % 2 == 1)
    def _():
        pltpu.make_async_copy(adj_hbm.at[pl.ds(r * BR, BR), :], buf1, sem1).wait()
        out_ref[...] = (
            jnp.dot(
                buf1[...].astype(jnp.bfloat16),
                support_ref[...],
                preferred_element_type=jnp.float32,
            )
            + bias_ref[...]
        )


@jax.jit
def kernel(feat, adj, weight, bias):
    bias2d = bias.reshape(1, D_OUT)
    grid = (NR,)
    out = pl.pallas_call(
        _gcn_kernel,
        grid=grid,
        in_specs=[
            pl.BlockSpec((D_IN, D_OUT), lambda r: (0, 0)),
            pl.BlockSpec((1, D_OUT), lambda r: (0, 0)),
            pl.BlockSpec(memory_space=pl.ANY),
            pl.BlockSpec(memory_space=pl.ANY),
        ],
        out_specs=pl.BlockSpec((BR, D_OUT), lambda r: (r, 0)),
        out_shape=jax.ShapeDtypeStruct((N, D_OUT), jnp.float32),
        scratch_shapes=[
            pltpu.VMEM((BR, N), jnp.float32),
            pltpu.VMEM((BR, N), jnp.float32),
            pltpu.VMEM((N, D_IN), jnp.float32),
            pltpu.VMEM((N, D_OUT), jnp.bfloat16),
            pltpu.SemaphoreType.DMA,
            pltpu.SemaphoreType.DMA,
            pltpu.SemaphoreType.DMA,
        ],
    )(weight, bias2d, feat, adj)
    return out


# speedup vs baseline: 1.0218x; 1.0218x over previous
import jax
import jax.numpy as jnp
from jax.experimental import pallas as pl
from jax.experimental.pallas import tpu as pltpu

N = 10000
D_IN = 128
D_OUT = 128
BR = 400


def _gcn_kernel(feat_ref, adj_ref, weight_ref, bias_ref, out_ref, support_ref):
    r = pl.program_id(0)

    @pl.when(r == 0)
    def _():
        support_ref[...] = jnp.dot(
            feat_ref[...], weight_ref[...], preferred_element_type=jnp.float32
        )

    out_ref[...] = (
        jnp.dot(adj_ref[...], support_ref[...], preferred_element_type=jnp.float32)
        + bias_ref[...]
    )


@jax.jit
def kernel(feat, adj, weight, bias):
    bias2d = bias.reshape(1, D_OUT)
    grid = (N // BR,)
    out = pl.pallas_call(
        _gcn_kernel,
        grid=grid,
        in_specs=[
            pl.BlockSpec((N, D_IN), lambda r: (0, 0)),
            pl.BlockSpec((BR, N), lambda r: (r, 0)),
            pl.BlockSpec((D_IN, D_OUT), lambda r: (0, 0)),
            pl.BlockSpec((1, D_OUT), lambda r: (0, 0)),
        ],
        out_specs=pl.BlockSpec((BR, D_OUT), lambda r: (r, 0)),
        out_shape=jax.ShapeDtypeStruct((N, D_OUT), jnp.float32),
        scratch_shapes=[pltpu.VMEM((N, D_OUT), jnp.float32)],
    )(feat, adj, weight, bias2d)
    return out
